# baseline (speedup 1.0000x reference)
import functools

import jax
import jax.numpy as jnp
import numpy as np
from jax import lax
from jax.experimental import pallas as pl
from jax.experimental.pallas import tpu as pltpu
from jax.experimental.pallas import tpu_sc as plsc

N_NODES = 10000
HIDDEN = 128
NEG_BIG = -1e30

EDGE_BLK = 512
NODE_BLK = 1000

SC_NC, SC_NS = 2, 16
SC_NW = SC_NC * SC_NS
GATHER_CH = 80
SEG_CH = 256
NPW = 320
N_PAD = SC_NW * NPW

SEG_CHK = 80
STG_N = 112
STG_DUMP = STG_N
SB_STASH = 127
STG_ROWS = 128
FLUSH_AT = STG_N - SEG_CHK
TRASH_FLUSH = 10080
TRASH_SB = 10040
FAKE_NODE = 10220


def _make_sc_gather256(ep):
    eb = ep // SC_NW
    assert ep % SC_NW == 0 and eb % GATHER_CH == 0
    nch = eb // GATHER_CH
    mesh = plsc.VectorSubcoreMesh(core_axis_name="c", subcore_axis_name="s")

    @functools.partial(
        pl.kernel,
        out_type=[
            jax.ShapeDtypeStruct((ep, 256), jnp.float32),
            jax.ShapeDtypeStruct((ep, 256), jnp.float32),
        ],
        mesh=mesh,
        scratch_types=[
            pltpu.VMEM((eb,), jnp.int32),
            pltpu.VMEM((eb,), jnp.int32),
            pltpu.VMEM((GATHER_CH, 256), jnp.float32),
            pltpu.VMEM((GATHER_CH, 256), jnp.float32),
            pltpu.VMEM((GATHER_CH, 256), jnp.float32),
            pltpu.VMEM((GATHER_CH, 256), jnp.float32),
            pltpu.SemaphoreType.DMA,
            pltpu.SemaphoreType.DMA,
        ],
    )
    def kg(t_hbm, srow_hbm, scol_hbm, gr_hbm, gc_hbm,
           ridx_v, cidx_v, rbuf0, cbuf0, rbuf1, cbuf1, sem0, sem1):
        wid = lax.axis_index("s") * SC_NC + lax.axis_index("c")
        base = wid * eb
        pltpu.sync_copy(srow_hbm.at[pl.ds(base, eb)], ridx_v)
        pltpu.sync_copy(scol_hbm.at[pl.ds(base, eb)], cidx_v)

        def gathers(c, rb, cb, sem):
            off = c * GATHER_CH
            cp_r = pltpu.make_async_copy(
                t_hbm.at[ridx_v.at[pl.ds(off, GATHER_CH)]], rb, sem)
            cp_c = pltpu.make_async_copy(
                t_hbm.at[cidx_v.at[pl.ds(off, GATHER_CH)]], cb, sem)
            return cp_r, cp_c

        def start(c, rb, cb, sem):
            cp_r, cp_c = gathers(c, rb, cb, sem)
            cp_r.start()
            cp_c.start()

        def drain(c, rb, cb, sem):
            cp_r, cp_c = gathers(c, rb, cb, sem)
            cp_r.wait()
            cp_c.wait()
            off = c * GATHER_CH
            pltpu.sync_copy(rb, gr_hbm.at[pl.ds(base + off, GATHER_CH)])
            pltpu.sync_copy(cb, gc_hbm.at[pl.ds(base + off, GATHER_CH)])

        start(0, rbuf0, cbuf0, sem0)

        def body(c, _):
            @pl.when(jnp.logical_and(c + 1 < nch, c % 2 == 0))
            def _():
                start(c + 1, rbuf1, cbuf1, sem1)

            @pl.when(jnp.logical_and(c + 1 < nch, c % 2 == 1))
            def _():
                start(c + 1, rbuf0, cbuf0, sem0)

            @pl.when(c % 2 == 0)
            def _():
                drain(c, rbuf0, cbuf0, sem0)

            @pl.when(c % 2 == 1)
            def _():
                drain(c, rbuf1, cbuf1, sem1)

            return ()

        lax.fori_loop(0, nch, body, ())

    return kg


def _make_sc_segmax_main(ep):
    eb = ep // SC_NW
    assert eb % SEG_CHK == 0
    nchk = eb // SEG_CHK
    mesh = plsc.VectorSubcoreMesh(core_axis_name="c", subcore_axis_name="s")

    @functools.partial(
        pl.kernel,
        out_type=[
            jax.ShapeDtypeStruct((N_PAD, 128), jnp.float32),
            jax.ShapeDtypeStruct((SC_NW * 16, 128), jnp.float32),
            jax.ShapeDtypeStruct((SC_NW * 16,), jnp.int32),
        ],
        mesh=mesh,
        scratch_types=[
            pltpu.VMEM((ep // SC_NW + 16,), jnp.int32),
            pltpu.VMEM((SEG_CHK, 128), jnp.float32),
            pltpu.VMEM((STG_ROWS, 128), jnp.float32),
            pltpu.VMEM((STG_ROWS,), jnp.int32),
            pltpu.VMEM((16, 128), jnp.float32),
            pltpu.VMEM((16,), jnp.int32),
            pltpu.VMEM((128,), jnp.float32),
            pltpu.SemaphoreType.DMA,
            pltpu.SemaphoreType.DMA,
        ],
    )
    def k3(m_hbm, srow_hbm, o_hbm, sb_hbm, sbid_hbm,
           srw_v, mbuf, stg, ids_v, sbbuf, sbid_v, runb, sem, sem2):
        wid = lax.axis_index("s") * SC_NC + lax.axis_index("c")
        base = pl.multiple_of(wid * eb, 8)
        pltpu.sync_copy(srow_hbm.at[pl.ds(base, eb + 16)], srw_v)
        lanes = lax.iota(jnp.int32, 16)

        minus_inf = jnp.full((16,), -jnp.inf, jnp.float32)
        for k in range(STG_ROWS // 16):
            ids_v[pl.ds(k * 16, 16)] = TRASH_FLUSH + k * 16 + lanes
        for f in range(8):
            runb[pl.ds(f * 16, 16)] = minus_inf
            stg[SB_STASH, pl.ds(f * 16, 16)] = minus_inf

        zeros16 = jnp.zeros((16, 1), jnp.int32)
        _gdn = lax.GatherDimensionNumbers(
            offset_dims=(), collapsed_slice_dims=(0,), start_index_map=(0,))

        def splat0(vec):
            return lax.gather(
                vec, zeros16, _gdn, slice_sizes=(1,),
                mode=lax.GatherScatterMode.PROMISE_IN_BOUNDS)

        curvec0 = splat0(srw_v[pl.ds(0, 16)])
        cur0 = srw_v[pl.ds(0, 16)][0]

        def chunk_body(c, carry):
            cur, closes, slot, curvec, sbid0v = carry[:5]
            runs = list(carry[5:])
            coff = pl.multiple_of(c * SEG_CHK, 8)
            pltpu.sync_copy(m_hbm.at[pl.ds(base + coff, SEG_CHK)], mbuf)

            def edge_body(j, st):
                cur_, closes_, slot_, curvec_, sbid0v_ = st[:5]
                runs_ = list(st[5:])
                ndv = srw_v[pl.ds(coff + j, 16)]
                nd = ndv[0]
                change = nd != cur_
                first_close = jnp.logical_and(change, closes_ == 0)
                norm_close = jnp.logical_and(change, closes_ > 0)

                row = jnp.where(
                    change, jnp.where(norm_close, slot_, SB_STASH),
                    STG_DUMP + (j % 15))
                new_runs = []
                for f in range(8):
                    sl = pl.ds(f * 16, 16)
                    stg[row, sl] = runb[sl]
                    mrow = mbuf[j, sl]
                    nr = jnp.where(change, mrow,
                                   jnp.maximum(runs_[f], mrow))
                    runb[sl] = nr
                    new_runs.append(nr)
                grp = (slot_ // 16) * 16
                idgrp = ids_v[pl.ds(grp, 16)]
                maskv = lanes == (slot_ % 16)
                inner = jnp.where(norm_close, curvec_, idgrp)
                ids_v[pl.ds(grp, 16)] = jnp.where(maskv, inner, idgrp)

                sbid0v_ = jnp.where(first_close, curvec_, sbid0v_)
                slot_ = jnp.where(norm_close, slot_ + 1, slot_)
                closes_ = jnp.where(change, closes_ + 1, closes_)
                cur_ = jnp.where(change, nd, cur_)
                curvec_ = jnp.where(change, splat0(ndv), curvec_)
                return (cur_, closes_, slot_, curvec_, sbid0v_, *new_runs)

            st = lax.fori_loop(
                0, SEG_CHK, edge_body,
                (cur, closes, slot, curvec, sbid0v, *runs))
            cur, closes, slot, curvec, sbid0v = st[:5]
            runs = list(st[5:])

            flush = slot >= FLUSH_AT

            @pl.when(flush)
            def _():
                pltpu.async_copy(stg, o_hbm.at[ids_v], sem).wait()

            for k in range(STG_ROWS // 16):
                old = ids_v[pl.ds(k * 16, 16)]
                trash = TRASH_FLUSH + k * 16 + lanes
                ids_v[pl.ds(k * 16, 16)] = jnp.where(flush, trash, old)
            slot = jnp.where(flush, 0, slot)
            return (cur, closes, slot, curvec, sbid0v, *runs)

        st = lax.fori_loop(
            0, nchk, chunk_body,
            (cur0, jnp.int32(0), jnp.int32(0), curvec0, curvec0,
             *([minus_inf] * 8)))
        cur, closes, slot, curvec, sbid0v = st[:5]

        pltpu.async_copy(stg, o_hbm.at[ids_v], sem).wait()
        for f in range(8):
            sl = pl.ds(f * 16, 16)
            sbbuf[0, sl] = stg[SB_STASH, sl]
            sbbuf[1, sl] = runb[sl]
        sbid0_effv = jnp.where(closes > 0, sbid0v, curvec)
        idvec = jnp.where(lanes == 0, sbid0_effv,
                          jnp.where(lanes == 1, curvec, TRASH_SB + lanes))
        sbid_v[pl.ds(0, 16)] = idvec
        sb_base = pl.multiple_of(wid * 16, 8)
        pltpu.sync_copy(sbid_v, sbid_hbm.at[pl.ds(sb_base, 16)])
        pltpu.sync_copy(sbbuf, sb_hbm.at[pl.ds(sb_base, 16)])

    return k3


def _make_sc_segmax_merge():
    mesh = plsc.VectorSubcoreMesh(core_axis_name="c", subcore_axis_name="s")
    nsb = SC_NW * 16

    @functools.partial(
        pl.kernel,
        out_type=jax.ShapeDtypeStruct((N_PAD, 128), jnp.float32),
        mesh=mesh,
        scratch_types=[
            pltpu.VMEM((nsb, 128), jnp.float32),
            pltpu.VMEM((nsb + 16,), jnp.int32),
            pltpu.VMEM((NPW + 16, 128), jnp.float32),
            pltpu.VMEM((128,), jnp.float32),
            pltpu.SemaphoreType.DMA,
        ],
    )
    def k3m(o_in_hbm, sb_hbm, sbid_hbm, o_hbm, sb_v, sbid_v, oblk, runb, sem):
        wid = lax.axis_index("s") * SC_NC + lax.axis_index("c")
        n0 = pl.multiple_of(wid * NPW, 8)
        pltpu.sync_copy(sb_hbm, sb_v)
        pltpu.sync_copy(sbid_hbm, sbid_v.at[pl.ds(0, nsb)])
        pltpu.sync_copy(o_in_hbm.at[pl.ds(n0, NPW)], oblk.at[pl.ds(0, NPW)])

        minus_inf = jnp.full((16,), -jnp.inf, jnp.float32)
        for f in range(8):
            runb[pl.ds(f * 16, 16)] = minus_inf

        def flush_row(do_it, node):
            rel = node - n0
            inr = jnp.logical_and(do_it,
                                  jnp.logical_and(rel >= 0, rel < NPW))
            row = jnp.where(inr, rel, NPW)
            for f in range(8):
                sl = pl.ds(f * 16, 16)
                run_f = runb[sl]
                old = oblk[row, sl]
                oblk[row, sl] = jnp.where(inr, run_f, old)

        def body(k, curn):
            w_k, s_k = k // 2, k % 2
            idx = w_k * 16 + s_k
            node = sbid_v[pl.ds(idx, 16)][0]
            change = node != curn
            flush_row(change, curn)
            for f in range(8):
                sl = pl.ds(f * 16, 16)
                row = sb_v[idx, sl]
                runb[sl] = jnp.where(change, row,
                                     jnp.maximum(runb[sl], row))
            return jnp.where(change, node, curn)

        first_node = sbid_v[pl.ds(0, 16)][0]
        last_node = lax.fori_loop(0, 2 * SC_NW, body, first_node)
        flush_row(jnp.bool_(True), last_node)
        pltpu.sync_copy(oblk.at[pl.ds(0, NPW)], o_hbm.at[pl.ds(n0, NPW)])

    return k3m


def _k0_body(vp_ref, wp_ref, bp_ref, wd_ref, t_ref, s1_ref):
    t0 = jnp.dot(vp_ref[...], wp_ref[...], preferred_element_type=jnp.float32)
    t0 = t0 + bp_ref[...]
    t_ref[...] = t0
    s1_ref[...] = jnp.dot(t0, wd_ref[...], preferred_element_type=jnp.float32)


def _node_init(verts_pad, W_pos_pad, b_pos, Wd0):
    n = verts_pad.shape[0]
    grid = (n // NODE_BLK,)
    return pl.pallas_call(
        _k0_body,
        grid=grid,
        in_specs=[
            pl.BlockSpec((NODE_BLK, 8), lambda i: (i, 0)),
            pl.BlockSpec((8, 256), lambda i: (0, 0)),
            pl.BlockSpec((1, 256), lambda i: (0, 0)),
            pl.BlockSpec((256, 128), lambda i: (0, 0)),
        ],
        out_specs=[
            pl.BlockSpec((NODE_BLK, 256), lambda i: (i, 0)),
            pl.BlockSpec((NODE_BLK, 128), lambda i: (i, 0)),
        ],
        out_shape=[
            jax.ShapeDtypeStruct((n, 256), jnp.float32),
            jax.ShapeDtypeStruct((n, 128), jnp.float32),
        ],
    )(verts_pad, W_pos_pad, b_pos, Wd0)


def _k1_body(seg_ref, s1p_ref, b1_ref, p_ref, wd_ref, keep_ref,
             t_ref, s1_ref):
    net = seg_ref[...] + b1_ref[...] + s1p_ref[...]
    net = jnp.where(keep_ref[...][:, :1] > 0, net, 0.0)
    pool = jnp.max(net, axis=1, keepdims=True)
    blk = net.shape[0]
    trec = jnp.concatenate(
        [net, pool, p_ref[...][:, :3], jnp.zeros((blk, 124), jnp.float32)],
        axis=1)
    t_ref[...] = trec
    s1_ref[...] = jnp.dot(trec, wd_ref[...], preferred_element_type=jnp.float32)


def _node_stage(seg, s1_prev, b1_prev, verts_pad, Wd_rec, keep):
    n = seg.shape[0]
    grid = (n // NODE_BLK,)
    return pl.pallas_call(
        _k1_body,
        grid=grid,
        in_specs=[
            pl.BlockSpec((NODE_BLK, 128), lambda i: (i, 0)),
            pl.BlockSpec((NODE_BLK, 128), lambda i: (i, 0)),
            pl.BlockSpec((1, 128), lambda i: (0, 0)),
            pl.BlockSpec((NODE_BLK, 8), lambda i: (i, 0)),
            pl.BlockSpec((256, 128), lambda i: (0, 0)),
            pl.BlockSpec((NODE_BLK, 8), lambda i: (i, 0)),
        ],
        out_specs=[
            pl.BlockSpec((NODE_BLK, 256), lambda i: (i, 0)),
            pl.BlockSpec((NODE_BLK, 128), lambda i: (i, 0)),
        ],
        out_shape=[
            jax.ShapeDtypeStruct((n, 256), jnp.float32),
            jax.ShapeDtypeStruct((n, 128), jnp.float32),
        ],
    )(seg, s1_prev, b1_prev, verts_pad, Wd_rec, keep)


def _k2_body(gr_ref, gc_ref, wi_ref, w2_ref, b0_ref, m_ref):
    gr = gr_ref[...]
    gc = gc_ref[...]
    a = jnp.concatenate(
        [jnp.maximum(gr, 0.0), gr, gc, jnp.abs(gc - gr)], axis=1)
    inner = jnp.dot(a, wi_ref[...], preferred_element_type=jnp.float32,
                    precision=jax.lax.Precision.HIGHEST)
    inner = jnp.maximum(inner + b0_ref[...], 0.0)
    a2 = jnp.concatenate([inner, gc], axis=1)
    m_ref[...] = jnp.dot(a2, w2_ref[...], preferred_element_type=jnp.float32,
                         precision=jax.lax.Precision.HIGHEST)


def _edge_stage(GR, GC, Winner, W2, b0):
    ep, rec = GR.shape
    grid = (ep // EDGE_BLK,)
    return pl.pallas_call(
        _k2_body,
        grid=grid,
        in_specs=[
            pl.BlockSpec((EDGE_BLK, rec), lambda i: (i, 0)),
            pl.BlockSpec((EDGE_BLK, rec), lambda i: (i, 0)),
            pl.BlockSpec((4 * rec, 128), lambda i: (0, 0)),
            pl.BlockSpec((rec + 128, 128), lambda i: (0, 0)),
            pl.BlockSpec((1, 128), lambda i: (0, 0)),
        ],
        out_specs=pl.BlockSpec((EDGE_BLK, 128), lambda i: (i, 0)),
        out_shape=jax.ShapeDtypeStruct((ep, 128), jnp.float32),
    )(GR, GC, Winner, W2, b0)


def _kf_body(seg_ref, s1_ref, b1_ref, wc_ref, bc_ref, keep_ref, c_ref):
    net = seg_ref[...] + b1_ref[...] + s1_ref[...]
    net = jnp.where(keep_ref[...][:, :1] > 0, net, 0.0)
    net = jnp.maximum(net, 0.0)
    c_ref[...] = jnp.dot(net, wc_ref[...],
                         preferred_element_type=jnp.float32) + bc_ref[...]


def _final_stage(seg, s1_prev, b1_prev, W_c, b_c, keep):
    n = seg.shape[0]
    grid = (n // NODE_BLK,)
    return pl.pallas_call(
        _kf_body,
        grid=grid,
        in_specs=[
            pl.BlockSpec((NODE_BLK, 128), lambda i: (i, 0)),
            pl.BlockSpec((NODE_BLK, 128), lambda i: (i, 0)),
            pl.BlockSpec((1, 128), lambda i: (0, 0)),
            pl.BlockSpec((128, 128), lambda i: (0, 0)),
            pl.BlockSpec((1, 128), lambda i: (0, 0)),
            pl.BlockSpec((NODE_BLK, 8), lambda i: (i, 0)),
        ],
        out_specs=pl.BlockSpec((NODE_BLK, 128), lambda i: (i, 0)),
        out_shape=jax.ShapeDtypeStruct((n, 128), jnp.float32),
    )(seg, s1_prev, b1_prev, W_c, b_c, keep)


def _to_rec(M):
    return jnp.concatenate([
        M[:128],
        jnp.sum(M[128:256], axis=0, keepdims=True),
        M[256:259],
        jnp.zeros((124, 128), jnp.float32),
    ], axis=0)


def _pack_layer(blk, first):
    d2 = 256 if first else 259
    W0, b0, W1, b1, Ws = blk["W0"], blk["b0"], blk["W1"], blk["b1"], blk["Ws"]
    W0a, W0b = W0[:d2], W0[d2:]
    Wsa, Wsb = Ws[:d2], Ws[d2:]
    conv = (lambda m: m) if first else _to_rec
    W0a_r, W0b_r = conv(W0a), conv(W0b)
    Winner = jnp.concatenate(
        [W0a_r, -0.5 * W0b_r, 0.5 * W0b_r, 0.5 * W0b_r], axis=0)
    W2 = jnp.concatenate([W1, conv(Wsb)], axis=0)
    Wd = conv(Wsa - Wsb)
    return Winner, W2, Wd, b0[None, :], b1[None, :]


def kernel(verts, faces, W_pos, b_pos, blocks, W_c, b_c):
    n = verts.shape[0]
    e = jnp.concatenate(
        [faces[:, [0, 1]], faces[:, [0, 2]], faces[:, [1, 2]]], axis=0)
    e = jnp.sort(e, axis=1)
    row = jnp.concatenate([e[:, 0], e[:, 1]])
    col = jnp.concatenate([e[:, 1], e[:, 0]])
    perm = jnp.argsort(row)
    srow = row[perm]
    scol = col[perm]
    E = srow.shape[0]
    EP = ((E + EDGE_BLK - 1) // EDGE_BLK) * EDGE_BLK
    srow_p = jnp.pad(srow, (0, EP - E))
    scol_p = jnp.pad(scol, (0, EP - E))
    srow_x = jnp.pad(srow, (0, EP - E + 16), constant_values=FAKE_NODE)
    off_n = jnp.searchsorted(srow, jnp.arange(n + 1, dtype=jnp.int32))
    keep = jnp.pad((off_n[1:] > off_n[:-1]).astype(jnp.float32)[:, None],
                   ((0, 0), (0, 7)))

    verts_pad = jnp.pad(verts, ((0, 0), (0, 5)))
    W_pos_pad = jnp.pad(W_pos, ((0, 5), (0, 0)))

    packs = [_pack_layer(blk, i == 0) for i, blk in enumerate(blocks)]

    kg256 = _make_sc_gather256(EP)
    k3 = _make_sc_segmax_main(EP)
    k3m = _make_sc_segmax_merge()

    T, s1 = _node_init(verts_pad, W_pos_pad, b_pos[None, :], packs[0][2])
    for l in range(5):
        Winner, W2, _, b0, b1 = packs[l]
        GR, GC = kg256(T, srow_p, scol_p)
        M = _edge_stage(GR, GC, Winner, W2, b0)
        O_dir, SB, SBID = k3(M, srow_x)
        seg = k3m(O_dir, SB, SBID)[:n]
        if l < 4:
            T, s1 = _node_stage(seg, s1, b1, verts_pad, packs[l + 1][2], keep)
        else:
            out = _final_stage(seg, s1, b1, W_c, b_c[None, :], keep)
    return out


# speedup vs baseline: 1.5534x; 1.5534x over previous
import functools

import jax
import jax.numpy as jnp
import numpy as np
from jax import lax
from jax.experimental import pallas as pl
from jax.experimental.pallas import tpu as pltpu
from jax.experimental.pallas import tpu_sc as plsc

N_NODES = 10000
HIDDEN = 128
NEG_BIG = -1e30

EDGE_BLK = 512
NODE_BLK = 1000

SC_NC, SC_NS = 2, 16
SC_NW = SC_NC * SC_NS
GATHER_CH = 80
SEG_CH = 256
NPW = 320
N_PAD = SC_NW * NPW

SEG_CHK = 80
STG_N = 112
STG_DUMP = STG_N
SB_STASH = 127
STG_ROWS = 128
FLUSH_AT = STG_N - SEG_CHK
TRASH_FLUSH = 10080
TRASH_SB = 10040
FAKE_NODE = 10220


def _make_sc_gather256(ep):
    eb = ep // SC_NW
    assert ep % SC_NW == 0 and eb % GATHER_CH == 0
    nch = eb // GATHER_CH
    mesh = plsc.VectorSubcoreMesh(core_axis_name="c", subcore_axis_name="s")

    @functools.partial(
        pl.kernel,
        out_type=[
            jax.ShapeDtypeStruct((ep, 256), jnp.float32),
            jax.ShapeDtypeStruct((ep, 256), jnp.float32),
        ],
        mesh=mesh,
        scratch_types=[
            pltpu.VMEM((eb,), jnp.int32),
            pltpu.VMEM((eb,), jnp.int32),
            pltpu.VMEM((GATHER_CH, 256), jnp.float32),
            pltpu.VMEM((GATHER_CH, 256), jnp.float32),
            pltpu.VMEM((GATHER_CH, 256), jnp.float32),
            pltpu.VMEM((GATHER_CH, 256), jnp.float32),
            pltpu.SemaphoreType.DMA,
            pltpu.SemaphoreType.DMA,
        ],
    )
    def kg(t_hbm, srow_hbm, scol_hbm, gr_hbm, gc_hbm,
           ridx_v, cidx_v, rbuf0, cbuf0, rbuf1, cbuf1, sem0, sem1):
        wid = lax.axis_index("s") * SC_NC + lax.axis_index("c")
        base = wid * eb
        pltpu.sync_copy(srow_hbm.at[pl.ds(base, eb)], ridx_v)
        pltpu.sync_copy(scol_hbm.at[pl.ds(base, eb)], cidx_v)

        def gathers(c, rb, cb, sem):
            off = c * GATHER_CH
            cp_r = pltpu.make_async_copy(
                t_hbm.at[ridx_v.at[pl.ds(off, GATHER_CH)]], rb, sem)
            cp_c = pltpu.make_async_copy(
                t_hbm.at[cidx_v.at[pl.ds(off, GATHER_CH)]], cb, sem)
            return cp_r, cp_c

        def start(c, rb, cb, sem):
            cp_r, cp_c = gathers(c, rb, cb, sem)
            cp_r.start()
            cp_c.start()

        def drain(c, rb, cb, sem):
            cp_r, cp_c = gathers(c, rb, cb, sem)
            cp_r.wait()
            cp_c.wait()
            off = c * GATHER_CH
            pltpu.sync_copy(rb, gr_hbm.at[pl.ds(base + off, GATHER_CH)])
            pltpu.sync_copy(cb, gc_hbm.at[pl.ds(base + off, GATHER_CH)])

        start(0, rbuf0, cbuf0, sem0)

        def body(c, _):
            @pl.when(jnp.logical_and(c + 1 < nch, c % 2 == 0))
            def _():
                start(c + 1, rbuf1, cbuf1, sem1)

            @pl.when(jnp.logical_and(c + 1 < nch, c % 2 == 1))
            def _():
                start(c + 1, rbuf0, cbuf0, sem0)

            @pl.when(c % 2 == 0)
            def _():
                drain(c, rbuf0, cbuf0, sem0)

            @pl.when(c % 2 == 1)
            def _():
                drain(c, rbuf1, cbuf1, sem1)

            return ()

        lax.fori_loop(0, nch, body, ())

    return kg


def _make_sc_segmax_main(ep):
    eb = ep // SC_NW
    assert eb % SEG_CHK == 0
    nchk = eb // SEG_CHK
    mesh = plsc.VectorSubcoreMesh(core_axis_name="c", subcore_axis_name="s")

    @functools.partial(
        pl.kernel,
        out_type=[
            jax.ShapeDtypeStruct((N_PAD, 128), jnp.float32),
            jax.ShapeDtypeStruct((SC_NW * 16, 128), jnp.float32),
            jax.ShapeDtypeStruct((SC_NW * 16,), jnp.int32),
        ],
        mesh=mesh,
        scratch_types=[
            pltpu.VMEM((ep // SC_NW + 16,), jnp.int32),
            pltpu.VMEM((SEG_CHK, 128), jnp.float32),
            pltpu.VMEM((STG_ROWS, 128), jnp.float32),
            pltpu.VMEM((STG_ROWS,), jnp.int32),
            pltpu.VMEM((16, 128), jnp.float32),
            pltpu.VMEM((16,), jnp.int32),
            pltpu.VMEM((128,), jnp.float32),
            pltpu.SemaphoreType.DMA,
            pltpu.SemaphoreType.DMA,
        ],
    )
    def k3(m_hbm, srow_hbm, o_hbm, sb_hbm, sbid_hbm,
           srw_v, mbuf, stg, ids_v, sbbuf, sbid_v, runb, sem, sem2):
        wid = lax.axis_index("s") * SC_NC + lax.axis_index("c")
        base = pl.multiple_of(wid * eb, 8)
        pltpu.sync_copy(srow_hbm.at[pl.ds(base, eb + 16)], srw_v)
        lanes = lax.iota(jnp.int32, 16)

        minus_inf = jnp.full((16,), -jnp.inf, jnp.float32)
        for k in range(STG_ROWS // 16):
            ids_v[pl.ds(k * 16, 16)] = TRASH_FLUSH + k * 16 + lanes
        for f in range(8):
            runb[pl.ds(f * 16, 16)] = minus_inf
            stg[SB_STASH, pl.ds(f * 16, 16)] = minus_inf

        zeros16 = jnp.zeros((16, 1), jnp.int32)
        _gdn = lax.GatherDimensionNumbers(
            offset_dims=(), collapsed_slice_dims=(0,), start_index_map=(0,))

        def splat0(vec):
            return lax.gather(
                vec, zeros16, _gdn, slice_sizes=(1,),
                mode=lax.GatherScatterMode.PROMISE_IN_BOUNDS)

        curvec0 = splat0(srw_v[pl.ds(0, 16)])
        cur0 = srw_v[pl.ds(0, 16)][0]

        def chunk_body(c, carry):
            cur, closes, slot, curvec, sbid0v = carry[:5]
            runs = list(carry[5:])
            coff = pl.multiple_of(c * SEG_CHK, 8)
            pltpu.sync_copy(m_hbm.at[pl.ds(base + coff, SEG_CHK)], mbuf)

            def edge_body(j, st):
                cur_, closes_, slot_, curvec_, sbid0v_ = st[:5]
                runs_ = list(st[5:])
                ndv = srw_v[pl.ds(coff + j, 16)]
                nd = ndv[0]
                change = nd != cur_
                first_close = jnp.logical_and(change, closes_ == 0)
                norm_close = jnp.logical_and(change, closes_ > 0)

                row = jnp.where(
                    change, jnp.where(norm_close, slot_, SB_STASH),
                    STG_DUMP + (j % 15))
                new_runs = []
                for f in range(8):
                    sl = pl.ds(f * 16, 16)
                    stg[row, sl] = runb[sl]
                    mrow = mbuf[j, sl]
                    nr = jnp.where(change, mrow,
                                   jnp.maximum(runs_[f], mrow))
                    runb[sl] = nr
                    new_runs.append(nr)
                grp = (slot_ // 16) * 16
                idgrp = ids_v[pl.ds(grp, 16)]
                maskv = lanes == (slot_ % 16)
                inner = jnp.where(norm_close, curvec_, idgrp)
                ids_v[pl.ds(grp, 16)] = jnp.where(maskv, inner, idgrp)

                sbid0v_ = jnp.where(first_close, curvec_, sbid0v_)
                slot_ = jnp.where(norm_close, slot_ + 1, slot_)
                closes_ = jnp.where(change, closes_ + 1, closes_)
                cur_ = jnp.where(change, nd, cur_)
                curvec_ = jnp.where(change, splat0(ndv), curvec_)
                return (cur_, closes_, slot_, curvec_, sbid0v_, *new_runs)

            st = lax.fori_loop(
                0, SEG_CHK, edge_body,
                (cur, closes, slot, curvec, sbid0v, *runs))
            cur, closes, slot, curvec, sbid0v = st[:5]
            runs = list(st[5:])

            flush = slot >= FLUSH_AT

            @pl.when(flush)
            def _():
                pltpu.async_copy(stg, o_hbm.at[ids_v], sem).wait()

            for k in range(STG_ROWS // 16):
                old = ids_v[pl.ds(k * 16, 16)]
                trash = TRASH_FLUSH + k * 16 + lanes
                ids_v[pl.ds(k * 16, 16)] = jnp.where(flush, trash, old)
            slot = jnp.where(flush, 0, slot)
            return (cur, closes, slot, curvec, sbid0v, *runs)

        st = lax.fori_loop(
            0, nchk, chunk_body,
            (cur0, jnp.int32(0), jnp.int32(0), curvec0, curvec0,
             *([minus_inf] * 8)))
        cur, closes, slot, curvec, sbid0v = st[:5]

        pltpu.async_copy(stg, o_hbm.at[ids_v], sem).wait()
        for f in range(8):
            sl = pl.ds(f * 16, 16)
            sbbuf[0, sl] = stg[SB_STASH, sl]
            sbbuf[1, sl] = runb[sl]
        sbid0_effv = jnp.where(closes > 0, sbid0v, curvec)
        idvec = jnp.where(lanes == 0, sbid0_effv,
                          jnp.where(lanes == 1, curvec, TRASH_SB + lanes))
        sbid_v[pl.ds(0, 16)] = idvec
        sb_base = pl.multiple_of(wid * 16, 8)
        pltpu.sync_copy(sbid_v, sbid_hbm.at[pl.ds(sb_base, 16)])
        pltpu.sync_copy(sbbuf, sb_hbm.at[pl.ds(sb_base, 16)])

    return k3


def _make_sc_segmax_merge():
    mesh = plsc.VectorSubcoreMesh(core_axis_name="c", subcore_axis_name="s")
    nsb = SC_NW * 16

    @functools.partial(
        pl.kernel,
        out_type=jax.ShapeDtypeStruct((N_PAD, 128), jnp.float32),
        mesh=mesh,
        scratch_types=[
            pltpu.VMEM((nsb, 128), jnp.float32),
            pltpu.VMEM((nsb + 16,), jnp.int32),
            pltpu.VMEM((NPW + 16, 128), jnp.float32),
            pltpu.VMEM((128,), jnp.float32),
            pltpu.SemaphoreType.DMA,
        ],
    )
    def k3m(o_in_hbm, sb_hbm, sbid_hbm, o_hbm, sb_v, sbid_v, oblk, runb, sem):
        wid = lax.axis_index("s") * SC_NC + lax.axis_index("c")
        n0 = pl.multiple_of(wid * NPW, 8)
        pltpu.sync_copy(sb_hbm, sb_v)
        pltpu.sync_copy(sbid_hbm, sbid_v.at[pl.ds(0, nsb)])
        pltpu.sync_copy(o_in_hbm.at[pl.ds(n0, NPW)], oblk.at[pl.ds(0, NPW)])

        minus_inf = jnp.full((16,), -jnp.inf, jnp.float32)
        for f in range(8):
            runb[pl.ds(f * 16, 16)] = minus_inf

        def flush_row(do_it, node):
            rel = node - n0
            inr = jnp.logical_and(do_it,
                                  jnp.logical_and(rel >= 0, rel < NPW))
            row = jnp.where(inr, rel, NPW)
            for f in range(8):
                sl = pl.ds(f * 16, 16)
                run_f = runb[sl]
                old = oblk[row, sl]
                oblk[row, sl] = jnp.where(inr, run_f, old)

        def body(k, curn):
            w_k, s_k = k // 2, k % 2
            idx = w_k * 16 + s_k
            node = sbid_v[pl.ds(idx, 16)][0]
            change = node != curn
            flush_row(change, curn)
            for f in range(8):
                sl = pl.ds(f * 16, 16)
                row = sb_v[idx, sl]
                runb[sl] = jnp.where(change, row,
                                     jnp.maximum(runb[sl], row))
            return jnp.where(change, node, curn)

        first_node = sbid_v[pl.ds(0, 16)][0]
        last_node = lax.fori_loop(0, 2 * SC_NW, body, first_node)
        flush_row(jnp.bool_(True), last_node)
        pltpu.sync_copy(oblk.at[pl.ds(0, NPW)], o_hbm.at[pl.ds(n0, NPW)])

    return k3m


def _k0_body(vp_ref, wp_ref, bp_ref, wd_ref, t_ref, s1_ref):
    t0 = jnp.dot(vp_ref[...], wp_ref[...], preferred_element_type=jnp.float32)
    t0 = t0 + bp_ref[...]
    t_ref[...] = t0
    s1_ref[...] = jnp.dot(t0, wd_ref[...], preferred_element_type=jnp.float32)


def _node_init(verts_pad, W_pos_pad, b_pos, Wd0):
    n = verts_pad.shape[0]
    grid = (n // NODE_BLK,)
    return pl.pallas_call(
        _k0_body,
        grid=grid,
        in_specs=[
            pl.BlockSpec((NODE_BLK, 8), lambda i: (i, 0)),
            pl.BlockSpec((8, 256), lambda i: (0, 0)),
            pl.BlockSpec((1, 256), lambda i: (0, 0)),
            pl.BlockSpec((256, 128), lambda i: (0, 0)),
        ],
        out_specs=[
            pl.BlockSpec((NODE_BLK, 256), lambda i: (i, 0)),
            pl.BlockSpec((NODE_BLK, 128), lambda i: (i, 0)),
        ],
        out_shape=[
            jax.ShapeDtypeStruct((n, 256), jnp.float32),
            jax.ShapeDtypeStruct((n, 128), jnp.float32),
        ],
    )(verts_pad, W_pos_pad, b_pos, Wd0)


def _k1_body(seg_ref, s1p_ref, b1_ref, p_ref, wd_ref, keep_ref,
             t_ref, s1_ref):
    net = seg_ref[...] + b1_ref[...] + s1p_ref[...]
    net = jnp.where(keep_ref[...][:, :1] > 0, net, 0.0)
    pool = jnp.max(net, axis=1, keepdims=True)
    blk = net.shape[0]
    trec = jnp.concatenate(
        [net, pool, p_ref[...][:, :3], jnp.zeros((blk, 124), jnp.float32)],
        axis=1)
    t_ref[...] = trec
    s1_ref[...] = jnp.dot(trec, wd_ref[...], preferred_element_type=jnp.float32)


def _node_stage(seg, s1_prev, b1_prev, verts_pad, Wd_rec, keep):
    n = seg.shape[0]
    grid = (n // NODE_BLK,)
    return pl.pallas_call(
        _k1_body,
        grid=grid,
        in_specs=[
            pl.BlockSpec((NODE_BLK, 128), lambda i: (i, 0)),
            pl.BlockSpec((NODE_BLK, 128), lambda i: (i, 0)),
            pl.BlockSpec((1, 128), lambda i: (0, 0)),
            pl.BlockSpec((NODE_BLK, 8), lambda i: (i, 0)),
            pl.BlockSpec((256, 128), lambda i: (0, 0)),
            pl.BlockSpec((NODE_BLK, 8), lambda i: (i, 0)),
        ],
        out_specs=[
            pl.BlockSpec((NODE_BLK, 256), lambda i: (i, 0)),
            pl.BlockSpec((NODE_BLK, 128), lambda i: (i, 0)),
        ],
        out_shape=[
            jax.ShapeDtypeStruct((n, 256), jnp.float32),
            jax.ShapeDtypeStruct((n, 128), jnp.float32),
        ],
    )(seg, s1_prev, b1_prev, verts_pad, Wd_rec, keep)


def _k2_body(gr_ref, gc_ref, wi_ref, w2_ref, b0_ref, m_ref):
    gr = gr_ref[...]
    gc = gc_ref[...]
    a = jnp.concatenate(
        [jnp.maximum(gr, 0.0), gr, gc, jnp.abs(gc - gr)], axis=1)
    inner = jnp.dot(a, wi_ref[...], preferred_element_type=jnp.float32)
    inner = jnp.maximum(inner + b0_ref[...], 0.0)
    a2 = jnp.concatenate([inner, gc], axis=1)
    m_ref[...] = jnp.dot(a2, w2_ref[...], preferred_element_type=jnp.float32)


def _edge_stage(GR, GC, Winner, W2, b0):
    ep, rec = GR.shape
    grid = (ep // EDGE_BLK,)
    return pl.pallas_call(
        _k2_body,
        grid=grid,
        in_specs=[
            pl.BlockSpec((EDGE_BLK, rec), lambda i: (i, 0)),
            pl.BlockSpec((EDGE_BLK, rec), lambda i: (i, 0)),
            pl.BlockSpec((4 * rec, 128), lambda i: (0, 0)),
            pl.BlockSpec((rec + 128, 128), lambda i: (0, 0)),
            pl.BlockSpec((1, 128), lambda i: (0, 0)),
        ],
        out_specs=pl.BlockSpec((EDGE_BLK, 128), lambda i: (i, 0)),
        out_shape=jax.ShapeDtypeStruct((ep, 128), jnp.float32),
    )(GR, GC, Winner, W2, b0)


def _kf_body(seg_ref, s1_ref, b1_ref, wc_ref, bc_ref, keep_ref, c_ref):
    net = seg_ref[...] + b1_ref[...] + s1_ref[...]
    net = jnp.where(keep_ref[...][:, :1] > 0, net, 0.0)
    net = jnp.maximum(net, 0.0)
    c_ref[...] = jnp.dot(net, wc_ref[...],
                         preferred_element_type=jnp.float32) + bc_ref[...]


def _final_stage(seg, s1_prev, b1_prev, W_c, b_c, keep):
    n = seg.shape[0]
    grid = (n // NODE_BLK,)
    return pl.pallas_call(
        _kf_body,
        grid=grid,
        in_specs=[
            pl.BlockSpec((NODE_BLK, 128), lambda i: (i, 0)),
            pl.BlockSpec((NODE_BLK, 128), lambda i: (i, 0)),
            pl.BlockSpec((1, 128), lambda i: (0, 0)),
            pl.BlockSpec((128, 128), lambda i: (0, 0)),
            pl.BlockSpec((1, 128), lambda i: (0, 0)),
            pl.BlockSpec((NODE_BLK, 8), lambda i: (i, 0)),
        ],
        out_specs=pl.BlockSpec((NODE_BLK, 128), lambda i: (i, 0)),
        out_shape=jax.ShapeDtypeStruct((n, 128), jnp.float32),
    )(seg, s1_prev, b1_prev, W_c, b_c, keep)


def _to_rec(M):
    return jnp.concatenate([
        M[:128],
        jnp.sum(M[128:256], axis=0, keepdims=True),
        M[256:259],
        jnp.zeros((124, 128), jnp.float32),
    ], axis=0)


def _pack_layer(blk, first):
    d2 = 256 if first else 259
    W0, b0, W1, b1, Ws = blk["W0"], blk["b0"], blk["W1"], blk["b1"], blk["Ws"]
    W0a, W0b = W0[:d2], W0[d2:]
    Wsa, Wsb = Ws[:d2], Ws[d2:]
    conv = (lambda m: m) if first else _to_rec
    W0a_r, W0b_r = conv(W0a), conv(W0b)
    Winner = jnp.concatenate(
        [W0a_r, -0.5 * W0b_r, 0.5 * W0b_r, 0.5 * W0b_r], axis=0)
    W2 = jnp.concatenate([W1, conv(Wsb)], axis=0)
    Wd = conv(Wsa - Wsb)
    return Winner, W2, Wd, b0[None, :], b1[None, :]


def kernel(verts, faces, W_pos, b_pos, blocks, W_c, b_c):
    n = verts.shape[0]
    e = jnp.concatenate(
        [faces[:, [0, 1]], faces[:, [0, 2]], faces[:, [1, 2]]], axis=0)
    e = jnp.sort(e, axis=1)
    row = jnp.concatenate([e[:, 0], e[:, 1]])
    col = jnp.concatenate([e[:, 1], e[:, 0]])
    perm = jnp.argsort(row)
    srow = row[perm]
    scol = col[perm]
    E = srow.shape[0]
    EP = ((E + EDGE_BLK - 1) // EDGE_BLK) * EDGE_BLK
    srow_p = jnp.pad(srow, (0, EP - E))
    scol_p = jnp.pad(scol, (0, EP - E))
    srow_x = jnp.pad(srow, (0, EP - E + 16), constant_values=FAKE_NODE)
    off_n = jnp.searchsorted(srow, jnp.arange(n + 1, dtype=jnp.int32))
    keep = jnp.pad((off_n[1:] > off_n[:-1]).astype(jnp.float32)[:, None],
                   ((0, 0), (0, 7)))

    verts_pad = jnp.pad(verts, ((0, 0), (0, 5)))
    W_pos_pad = jnp.pad(W_pos, ((0, 5), (0, 0)))

    packs = [_pack_layer(blk, i == 0) for i, blk in enumerate(blocks)]

    kg256 = _make_sc_gather256(EP)
    k3 = _make_sc_segmax_main(EP)
    k3m = _make_sc_segmax_merge()

    T, s1 = _node_init(verts_pad, W_pos_pad, b_pos[None, :], packs[0][2])
    for l in range(5):
        Winner, W2, _, b0, b1 = packs[l]
        GR, GC = kg256(T, srow_p, scol_p)
        M = _edge_stage(GR, GC, Winner, W2, b0)
        O_dir, SB, SBID = k3(M, srow_x)
        seg = k3m(O_dir, SB, SBID)[:n]
        if l < 4:
            T, s1 = _node_stage(seg, s1, b1, verts_pad, packs[l + 1][2], keep)
        else:
            out = _final_stage(seg, s1, b1, W_c, b_c[None, :], keep)
    return out
